# baseline (speedup 1.0000x reference)
import functools

import jax
import jax.numpy as jnp
from jax import lax
from jax.experimental import pallas as pl
from jax.experimental.pallas import tpu as pltpu
from jax.experimental.pallas import tpu_sc as plsc

NN = 10000
NE = 320000
D = 128
NC = 2
NS = 16
ER = NE // D
ERH = ER // NC
ROWS_MAX = (ERH + NS - 1) // NS
NPAD = 10240
NZT = 624

_mesh = plsc.VectorSubcoreMesh(
    core_axis_name="c", subcore_axis_name="s", num_cores=NC, num_subcores=NS)


def _nrows(sid):
    return jnp.where(sid < ERH % NS, ROWS_MAX, ERH // NS)


def _deg_body(dst_hbm, ew_hbm, out_hbm, dstv, ewv, zv, acc, sem):
    del sem
    cid = lax.axis_index("c")
    sid = lax.axis_index("s")

    def zero(i, _):
        zv[pl.ds(i * 16, 16)] = jnp.zeros((16,), jnp.float32)
        return ()
    lax.fori_loop(0, 40, zero, ())
    pltpu.sync_copy(zv, acc.at[pl.ds(sid * 640, 640)])
    plsc.subcore_barrier()

    n = _nrows(sid)

    def body(i, _):
        @pl.when(i < n)
        def _():
            r = cid * ERH + i * NS + sid
            pltpu.sync_copy(dst_hbm.at[r], dstv)
            pltpu.sync_copy(ew_hbm.at[r], ewv)
            pltpu.sync_copy(ewv, acc.at[dstv], add=True)
        return ()
    lax.fori_loop(0, ROWS_MAX, body, ())

    plsc.subcore_barrier()
    pltpu.sync_copy(acc.at[pl.ds(sid * 640, 640)],
                    out_hbm.at[cid, pl.ds(sid * 640, 640)])


_deg_call = pl.kernel(
    _deg_body,
    out_type=jax.ShapeDtypeStruct((NC, NPAD), jnp.float32),
    mesh=_mesh,
    scratch_types=[
        pltpu.VMEM((D,), jnp.int32),
        pltpu.VMEM((D,), jnp.float32),
        pltpu.VMEM((640,), jnp.float32),
        pltpu.VMEM_SHARED((NPAD,), jnp.float32),
        pltpu.SemaphoreType.DMA,
    ],
)


NQ = ER // 8
NW = NC * NS
QMAX = (NQ + NW - 1) // NW
NTAIL = ER - NQ * 8
_DO_MUL = True
_DO_SCAT = True


def _prop_body(g_hbm, srcq_hbm, dstq_hbm, ewq_hbm, st_hbm, dt_hbm, et_hbm,
               out_hbm, srcv, dstv, ewv, rows, acc, gsem, isem):
    cid = lax.axis_index("c")
    sid = lax.axis_index("s")
    w = sid * NC + cid
    nq = jnp.where(w < NQ % NW, QMAX, NQ // NW)

    def iload(kb):
        s = kb % 2
        q = kb * NW + w
        pltpu.async_copy(srcq_hbm.at[q], srcv.at[s], isem)
        pltpu.async_copy(dstq_hbm.at[q], dstv.at[s], isem)
        pltpu.async_copy(ewq_hbm.at[q], ewv.at[s], isem)

    def iwait(kb):
        s = kb % 2
        q = kb * NW + w
        pltpu.make_async_copy(srcq_hbm.at[q], srcv.at[s], isem).wait()
        pltpu.make_async_copy(dstq_hbm.at[q], dstv.at[s], isem).wait()
        pltpu.make_async_copy(ewq_hbm.at[q], ewv.at[s], isem).wait()

    iload(0)

    def zero(i, _):
        for t in range(8):
            rows[0, i, pl.ds(t * 16, 16)] = jnp.zeros((16,), jnp.float32)
        return ()
    lax.fori_loop(0, D, zero, ())
    for c in range(4):
        pltpu.sync_copy(rows.at[0, pl.ds(0, 128)],
                        acc.at[pl.ds(sid * NZT + c * 128, 128)])
    pltpu.sync_copy(rows.at[0, pl.ds(0, 112)],
                    acc.at[pl.ds(sid * NZT + 512, 112)])

    @pl.when(sid == 0)
    def _():
        pltpu.sync_copy(rows.at[0, pl.ds(0, 16)], acc.at[pl.ds(NS * NZT, 16)])

    iwait(0)
    plsc.subcore_barrier()

    def gather_start(s, j, b):
        pltpu.async_copy(g_hbm.at[srcv.at[s, j]], rows.at[b], gsem)

    def gather_wait(b):
        pltpu.make_async_copy(g_hbm.at[srcv.at[0, 0]], rows.at[b],
                              gsem).wait()

    def multiply(s, j, b):
        @plsc.parallel_loop(0, 8, unroll=2)
        def _(g):
            wv = ewv[s, j, pl.ds(g * 16, 16)]
            for jj in range(16):
                w_ = wv[jj]
                k = g * 16 + jj
                for t in range(8):
                    rows[b, k, pl.ds(t * 16, 16)] = (
                        rows[b, k, pl.ds(t * 16, 16)] * w_)

    gather_start(0, 0, 0)

    def batch(kb, _):
        @pl.when(kb < nq)
        def _():
            bc = kb % 2

            @pl.when(kb + 1 < nq)
            def _():
                iload(kb + 1)

            def body(j, _):
                b = j % 2
                gather_wait(b)

                @pl.when(j < 7)
                def _():
                    gather_start(bc, j + 1, 1 - b)

                @pl.when(jnp.logical_and(j == 7, kb + 1 < nq))
                def _():
                    iwait(kb + 1)
                    gather_start(1 - bc, 0, 1 - b)
                if _DO_MUL:
                    multiply(bc, j, b)
                if _DO_SCAT:
                    pltpu.sync_copy(rows.at[b], acc.at[dstv.at[bc, j]],
                                    add=True)
                return ()
            lax.fori_loop(0, 8, body, ())
        return ()
    lax.fori_loop(0, QMAX, batch, ())

    @pl.when(w < NTAIL)
    def _():
        pltpu.sync_copy(st_hbm.at[w], srcv.at[0, 0])
        pltpu.sync_copy(dt_hbm.at[w], dstv.at[0, 0])
        pltpu.sync_copy(et_hbm.at[w], ewv.at[0, 0])
        pltpu.async_copy(g_hbm.at[srcv.at[0, 0]], rows.at[0], gsem).wait()
        multiply(0, 0, 0)
        pltpu.sync_copy(rows.at[0], acc.at[dstv.at[0, 0]], add=True)

    plsc.subcore_barrier()
    pltpu.sync_copy(acc.at[pl.ds(sid * NZT, NZT)],
                    out_hbm.at[cid, pl.ds(sid * NZT, NZT)])

    @pl.when(sid == 0)
    def _():
        pltpu.sync_copy(acc.at[pl.ds(NS * NZT, 16)],
                        out_hbm.at[cid, pl.ds(NS * NZT, 16)])


_prop_call = pl.kernel(
    _prop_body,
    out_type=jax.ShapeDtypeStruct((NC, NN, D), jnp.float32),
    mesh=_mesh,
    scratch_types=[
        pltpu.VMEM((2, 8, D), jnp.int32),
        pltpu.VMEM((2, 8, D), jnp.int32),
        pltpu.VMEM((2, 8, D), jnp.float32),
        pltpu.VMEM((2, D, D), jnp.float32),
        pltpu.VMEM_SHARED((NN, D), jnp.float32),
        pltpu.SemaphoreType.DMA,
        pltpu.SemaphoreType.DMA,
    ],
)


_NB = 1000
_GRID = NN // _NB


def _tc1_body(degp_ref, x_ref, w1_ref, dinv_ref, g1_ref):
    deg = 1.0 + degp_ref[:, 0] + degp_ref[:, 1]
    r = lax.rsqrt(jnp.maximum(deg, 1e-12))
    r = jnp.where(deg > 0, r, 0.0)[:, None]
    dinv_ref[...] = r
    h = jnp.dot(x_ref[...], w1_ref[...], preferred_element_type=jnp.float32)
    g1_ref[...] = h * r


def _tc2_body(acc_ref, g1_ref, dinv_ref, b1_ref, w2_ref, g2_ref):
    r = dinv_ref[...]
    h = (acc_ref[0] + acc_ref[1] + g1_ref[...]) * r + b1_ref[...][None, :]
    z = jnp.maximum(h, 0.0)
    g2_ref[...] = jnp.dot(z, w2_ref[...],
                          preferred_element_type=jnp.float32) * r


def _tc3_body(acc_ref, g2_ref, dinv_ref, b2_ref, wl_ref, bl_ref, out_ref):
    h = (acc_ref[0] + acc_ref[1] + g2_ref[...]) * dinv_ref[...] \
        + b2_ref[...][None, :]
    out_ref[...] = jnp.dot(h, wl_ref[...],
                           preferred_element_type=jnp.float32) \
        + bl_ref[...][None, :]


_node_spec = pl.BlockSpec((_NB, D), lambda i: (i, 0))
_dinv_spec = pl.BlockSpec((_NB, 1), lambda i: (i, 0))
_w_spec = pl.BlockSpec((D, D), lambda i: (0, 0))
_b_spec = pl.BlockSpec((D,), lambda i: (0,))
_acc_spec = pl.BlockSpec((NC, _NB, D), lambda i: (0, i, 0))

_tc1_call = pl.pallas_call(
    _tc1_body,
    grid=(_GRID,),
    in_specs=[pl.BlockSpec((_NB, NC), lambda i: (i, 0)), _node_spec, _w_spec],
    out_specs=[_dinv_spec, _node_spec],
    out_shape=[jax.ShapeDtypeStruct((NN, 1), jnp.float32),
               jax.ShapeDtypeStruct((NN, D), jnp.float32)],
)

_tc2_call = pl.pallas_call(
    _tc2_body,
    grid=(_GRID,),
    in_specs=[_acc_spec, _node_spec, _dinv_spec, _b_spec, _w_spec],
    out_specs=_node_spec,
    out_shape=jax.ShapeDtypeStruct((NN, D), jnp.float32),
)

_tc3_call = pl.pallas_call(
    _tc3_body,
    grid=(_GRID,),
    in_specs=[_acc_spec, _node_spec, _dinv_spec, _b_spec, _w_spec, _b_spec],
    out_specs=_node_spec,
    out_shape=jax.ShapeDtypeStruct((NN, D), jnp.float32),
)


def kernel(x, edge_index, edge_weight, W1, b1, W2, b2, Wl, bl):
    ei = edge_index.astype(jnp.int32)
    src = ei[0].reshape(ER, D)
    dst = ei[1].reshape(ER, D)
    ew = edge_weight.astype(jnp.float32).reshape(ER, D)
    srcq = src[:NQ * 8].reshape(NQ, 8, D)
    dstq = dst[:NQ * 8].reshape(NQ, 8, D)
    ewq = ew[:NQ * 8].reshape(NQ, 8, D)
    st, dt, et = src[NQ * 8:], dst[NQ * 8:], ew[NQ * 8:]

    degp = _deg_call(dst, ew)[:, :NN].T
    dinv, g1 = _tc1_call(degp, x, W1)
    acc1 = _prop_call(g1, srcq, dstq, ewq, st, dt, et)
    g2 = _tc2_call(acc1, g1, dinv, b1, W2)
    acc2 = _prop_call(g2, srcq, dstq, ewq, st, dt, et)
    return _tc3_call(acc2, g2, dinv, b2, Wl, bl)


# speedup vs baseline: 1.0474x; 1.0474x over previous
import functools

import jax
import jax.numpy as jnp
from jax import lax
from jax.experimental import pallas as pl
from jax.experimental.pallas import tpu as pltpu
from jax.experimental.pallas import tpu_sc as plsc

NN = 10000
NE = 320000
D = 128
NC = 2
NS = 16
ER = NE // D
ERH = ER // NC
ROWS_MAX = (ERH + NS - 1) // NS
NPAD = 10240
NZT = 624

_mesh = plsc.VectorSubcoreMesh(
    core_axis_name="c", subcore_axis_name="s", num_cores=NC, num_subcores=NS)


def _nrows(sid):
    return jnp.where(sid < ERH % NS, ROWS_MAX, ERH // NS)


def _deg_body(dstq_hbm, ewq_hbm, dt_hbm, et_hbm, out_hbm,
              dstv, ewv, zv, acc, isem, ssem):
    cid = lax.axis_index("c")
    sid = lax.axis_index("s")
    w = sid * NC + cid
    nq = jnp.where(w < NQ % NW, QMAX, NQ // NW)

    def iload(kb):
        s = kb % 2
        q = kb * NW + w
        pltpu.async_copy(dstq_hbm.at[q], dstv.at[s], isem)
        pltpu.async_copy(ewq_hbm.at[q], ewv.at[s], isem)

    def iwait(kb):
        s = kb % 2
        q = kb * NW + w
        pltpu.make_async_copy(dstq_hbm.at[q], dstv.at[s], isem).wait()
        pltpu.make_async_copy(ewq_hbm.at[q], ewv.at[s], isem).wait()

    iload(0)

    def zero(i, _):
        zv[pl.ds(i * 16, 16)] = jnp.zeros((16,), jnp.float32)
        return ()
    lax.fori_loop(0, 40, zero, ())
    pltpu.sync_copy(zv, acc.at[pl.ds(sid * 640, 640)])
    iwait(0)
    plsc.subcore_barrier()

    def body(kb, _):
        @pl.when(kb < nq)
        def _():
            s = kb % 2

            @pl.when(kb + 1 < nq)
            def _():
                iload(kb + 1)
            for j in range(8):
                pltpu.async_copy(ewv.at[s, j], acc.at[dstv.at[s, j]], ssem,
                                 add=True)
            for j in range(8):
                pltpu.make_async_copy(ewv.at[s, j], acc.at[dstv.at[s, j]],
                                      ssem).wait()

            @pl.when(kb + 1 < nq)
            def _():
                iwait(kb + 1)
        return ()
    lax.fori_loop(0, QMAX, body, ())

    @pl.when(w < NTAIL)
    def _():
        pltpu.sync_copy(dt_hbm.at[w], dstv.at[0, 0])
        pltpu.sync_copy(et_hbm.at[w], ewv.at[0, 0])
        pltpu.sync_copy(ewv.at[0, 0], acc.at[dstv.at[0, 0]], add=True)

    plsc.subcore_barrier()
    pltpu.sync_copy(acc.at[pl.ds(sid * 640, 640)],
                    out_hbm.at[cid, pl.ds(sid * 640, 640)])


_deg_call = pl.kernel(
    _deg_body,
    out_type=jax.ShapeDtypeStruct((NC, NPAD), jnp.float32),
    mesh=_mesh,
    scratch_types=[
        pltpu.VMEM((2, 8, D), jnp.int32),
        pltpu.VMEM((2, 8, D), jnp.float32),
        pltpu.VMEM((640,), jnp.float32),
        pltpu.VMEM_SHARED((NPAD,), jnp.float32),
        pltpu.SemaphoreType.DMA,
        pltpu.SemaphoreType.DMA,
    ],
)


NQ = ER // 8
NW = NC * NS
QMAX = (NQ + NW - 1) // NW
NTAIL = ER - NQ * 8
_DO_MUL = True
_DO_SCAT = True


def _prop_body(g_hbm, srcq_hbm, dstq_hbm, ewq_hbm, st_hbm, dt_hbm, et_hbm,
               out_hbm, srcv, dstv, ewv, rows, acc, gsem, isem):
    cid = lax.axis_index("c")
    sid = lax.axis_index("s")
    w = sid * NC + cid
    nq = jnp.where(w < NQ % NW, QMAX, NQ // NW)

    def iload(kb):
        s = kb % 2
        q = kb * NW + w
        pltpu.async_copy(srcq_hbm.at[q], srcv.at[s], isem)
        pltpu.async_copy(dstq_hbm.at[q], dstv.at[s], isem)
        pltpu.async_copy(ewq_hbm.at[q], ewv.at[s], isem)

    def iwait(kb):
        s = kb % 2
        q = kb * NW + w
        pltpu.make_async_copy(srcq_hbm.at[q], srcv.at[s], isem).wait()
        pltpu.make_async_copy(dstq_hbm.at[q], dstv.at[s], isem).wait()
        pltpu.make_async_copy(ewq_hbm.at[q], ewv.at[s], isem).wait()

    iload(0)

    def zero(i, _):
        for t in range(8):
            rows[0, i, pl.ds(t * 16, 16)] = jnp.zeros((16,), jnp.float32)
        return ()
    lax.fori_loop(0, D, zero, ())
    for c in range(4):
        pltpu.sync_copy(rows.at[0, pl.ds(0, 128)],
                        acc.at[pl.ds(sid * NZT + c * 128, 128)])
    pltpu.sync_copy(rows.at[0, pl.ds(0, 112)],
                    acc.at[pl.ds(sid * NZT + 512, 112)])

    @pl.when(sid == 0)
    def _():
        pltpu.sync_copy(rows.at[0, pl.ds(0, 16)], acc.at[pl.ds(NS * NZT, 16)])

    iwait(0)
    plsc.subcore_barrier()

    def gather_start(s, j, b):
        pltpu.async_copy(g_hbm.at[srcv.at[s, j]], rows.at[b], gsem)

    def gather_wait(b):
        pltpu.make_async_copy(g_hbm.at[srcv.at[0, 0]], rows.at[b],
                              gsem).wait()

    def multiply(s, j, b):
        @plsc.parallel_loop(0, 8, unroll=4)
        def _(g):
            wv = ewv[s, j, pl.ds(g * 16, 16)]
            for jj in range(16):
                w_ = wv[jj]
                k = g * 16 + jj
                for t in range(8):
                    rows[b, k, pl.ds(t * 16, 16)] = (
                        rows[b, k, pl.ds(t * 16, 16)] * w_)

    gather_start(0, 0, 0)

    def batch(kb, _):
        @pl.when(kb < nq)
        def _():
            bc = kb % 2

            @pl.when(kb + 1 < nq)
            def _():
                iload(kb + 1)

            def body(j, _):
                b = j % 2
                gather_wait(b)

                @pl.when(j < 7)
                def _():
                    gather_start(bc, j + 1, 1 - b)

                @pl.when(jnp.logical_and(j == 7, kb + 1 < nq))
                def _():
                    iwait(kb + 1)
                    gather_start(1 - bc, 0, 1 - b)
                if _DO_MUL:
                    multiply(bc, j, b)
                if _DO_SCAT:
                    pltpu.sync_copy(rows.at[b], acc.at[dstv.at[bc, j]],
                                    add=True)
                return ()
            lax.fori_loop(0, 8, body, ())
        return ()
    lax.fori_loop(0, QMAX, batch, ())

    @pl.when(w < NTAIL)
    def _():
        pltpu.sync_copy(st_hbm.at[w], srcv.at[0, 0])
        pltpu.sync_copy(dt_hbm.at[w], dstv.at[0, 0])
        pltpu.sync_copy(et_hbm.at[w], ewv.at[0, 0])
        pltpu.async_copy(g_hbm.at[srcv.at[0, 0]], rows.at[0], gsem).wait()
        multiply(0, 0, 0)
        pltpu.sync_copy(rows.at[0], acc.at[dstv.at[0, 0]], add=True)

    plsc.subcore_barrier()
    pltpu.sync_copy(acc.at[pl.ds(sid * NZT, NZT)],
                    out_hbm.at[cid, pl.ds(sid * NZT, NZT)])

    @pl.when(sid == 0)
    def _():
        pltpu.sync_copy(acc.at[pl.ds(NS * NZT, 16)],
                        out_hbm.at[cid, pl.ds(NS * NZT, 16)])


_prop_call = pl.kernel(
    _prop_body,
    out_type=jax.ShapeDtypeStruct((NC, NN, D), jnp.float32),
    mesh=_mesh,
    scratch_types=[
        pltpu.VMEM((2, 8, D), jnp.int32),
        pltpu.VMEM((2, 8, D), jnp.int32),
        pltpu.VMEM((2, 8, D), jnp.float32),
        pltpu.VMEM((2, D, D), jnp.float32),
        pltpu.VMEM_SHARED((NN, D), jnp.float32),
        pltpu.SemaphoreType.DMA,
        pltpu.SemaphoreType.DMA,
    ],
)


_NB = 1000
_GRID = NN // _NB


def _tc1_body(degp_ref, x_ref, w1_ref, dinv_ref, g1_ref):
    deg = 1.0 + degp_ref[:, 0] + degp_ref[:, 1]
    r = lax.rsqrt(jnp.maximum(deg, 1e-12))
    r = jnp.where(deg > 0, r, 0.0)[:, None]
    dinv_ref[...] = r
    h = jnp.dot(x_ref[...], w1_ref[...], preferred_element_type=jnp.float32)
    g1_ref[...] = h * r


def _tc2_body(acc_ref, g1_ref, dinv_ref, b1_ref, w2_ref, g2_ref):
    r = dinv_ref[...]
    h = (acc_ref[0] + acc_ref[1] + g1_ref[...]) * r + b1_ref[...][None, :]
    z = jnp.maximum(h, 0.0)
    g2_ref[...] = jnp.dot(z, w2_ref[...],
                          preferred_element_type=jnp.float32) * r


def _tc3_body(acc_ref, g2_ref, dinv_ref, b2_ref, wl_ref, bl_ref, out_ref):
    h = (acc_ref[0] + acc_ref[1] + g2_ref[...]) * dinv_ref[...] \
        + b2_ref[...][None, :]
    out_ref[...] = jnp.dot(h, wl_ref[...],
                           preferred_element_type=jnp.float32) \
        + bl_ref[...][None, :]


_node_spec = pl.BlockSpec((_NB, D), lambda i: (i, 0))
_dinv_spec = pl.BlockSpec((_NB, 1), lambda i: (i, 0))
_w_spec = pl.BlockSpec((D, D), lambda i: (0, 0))
_b_spec = pl.BlockSpec((D,), lambda i: (0,))
_acc_spec = pl.BlockSpec((NC, _NB, D), lambda i: (0, i, 0))

_tc1_call = pl.pallas_call(
    _tc1_body,
    grid=(_GRID,),
    in_specs=[pl.BlockSpec((_NB, NC), lambda i: (i, 0)), _node_spec, _w_spec],
    out_specs=[_dinv_spec, _node_spec],
    out_shape=[jax.ShapeDtypeStruct((NN, 1), jnp.float32),
               jax.ShapeDtypeStruct((NN, D), jnp.float32)],
)

_tc2_call = pl.pallas_call(
    _tc2_body,
    grid=(_GRID,),
    in_specs=[_acc_spec, _node_spec, _dinv_spec, _b_spec, _w_spec],
    out_specs=_node_spec,
    out_shape=jax.ShapeDtypeStruct((NN, D), jnp.float32),
)

_tc3_call = pl.pallas_call(
    _tc3_body,
    grid=(_GRID,),
    in_specs=[_acc_spec, _node_spec, _dinv_spec, _b_spec, _w_spec, _b_spec],
    out_specs=_node_spec,
    out_shape=jax.ShapeDtypeStruct((NN, D), jnp.float32),
)


def kernel(x, edge_index, edge_weight, W1, b1, W2, b2, Wl, bl):
    ei = edge_index.astype(jnp.int32)
    src = ei[0].reshape(ER, D)
    dst = ei[1].reshape(ER, D)
    ew = edge_weight.astype(jnp.float32).reshape(ER, D)
    srcq = src[:NQ * 8].reshape(NQ, 8, D)
    dstq = dst[:NQ * 8].reshape(NQ, 8, D)
    ewq = ew[:NQ * 8].reshape(NQ, 8, D)
    st, dt, et = src[NQ * 8:], dst[NQ * 8:], ew[NQ * 8:]

    degp = _deg_call(dstq, ewq, dt, et)[:, :NN].T
    dinv, g1 = _tc1_call(degp, x, W1)
    acc1 = _prop_call(g1, srcq, dstq, ewq, st, dt, et)
    g2 = _tc2_call(acc1, g1, dinv, b1, W2)
    acc2 = _prop_call(g2, srcq, dstq, ewq, st, dt, et)
    return _tc3_call(acc2, g2, dinv, b2, Wl, bl)


# speedup vs baseline: 1.9963x; 1.9059x over previous
import functools

import jax
import jax.numpy as jnp
from jax import lax
from jax.experimental import pallas as pl
from jax.experimental.pallas import tpu as pltpu
from jax.experimental.pallas import tpu_sc as plsc

NN = 10000
NE = 320000
D = 128
NC = 2
NS = 16
ER = NE // D
ERH = ER // NC
ROWS_MAX = (ERH + NS - 1) // NS
NPAD = 10240
NZT = 624

_mesh = plsc.VectorSubcoreMesh(
    core_axis_name="c", subcore_axis_name="s", num_cores=NC, num_subcores=NS)


def _nrows(sid):
    return jnp.where(sid < ERH % NS, ROWS_MAX, ERH // NS)


def _deg_body(dstq_hbm, ewq_hbm, dt_hbm, et_hbm, out_hbm,
              dstv, ewv, zv, acc, isem, ssem):
    cid = lax.axis_index("c")
    sid = lax.axis_index("s")
    w = sid * NC + cid
    nq = jnp.where(w < NQ % NW, QMAX, NQ // NW)

    def iload(kb):
        s = kb % 2
        q = kb * NW + w
        pltpu.async_copy(dstq_hbm.at[q], dstv.at[s], isem)
        pltpu.async_copy(ewq_hbm.at[q], ewv.at[s], isem)

    def iwait(kb):
        s = kb % 2
        q = kb * NW + w
        pltpu.make_async_copy(dstq_hbm.at[q], dstv.at[s], isem).wait()
        pltpu.make_async_copy(ewq_hbm.at[q], ewv.at[s], isem).wait()

    iload(0)

    def zero(i, _):
        zv[pl.ds(i * 16, 16)] = jnp.zeros((16,), jnp.float32)
        return ()
    lax.fori_loop(0, 40, zero, ())
    pltpu.sync_copy(zv, acc.at[pl.ds(sid * 640, 640)])
    iwait(0)
    plsc.subcore_barrier()

    def body(kb, _):
        @pl.when(kb < nq)
        def _():
            s = kb % 2

            @pl.when(kb + 1 < nq)
            def _():
                iload(kb + 1)
            for j in range(8):
                pltpu.async_copy(ewv.at[s, j], acc.at[dstv.at[s, j]], ssem,
                                 add=True)
            for j in range(8):
                pltpu.make_async_copy(ewv.at[s, j], acc.at[dstv.at[s, j]],
                                      ssem).wait()

            @pl.when(kb + 1 < nq)
            def _():
                iwait(kb + 1)
        return ()
    lax.fori_loop(0, QMAX, body, ())

    @pl.when(w < NTAIL)
    def _():
        pltpu.sync_copy(dt_hbm.at[w], dstv.at[0, 0])
        pltpu.sync_copy(et_hbm.at[w], ewv.at[0, 0])
        pltpu.sync_copy(ewv.at[0, 0], acc.at[dstv.at[0, 0]], add=True)

    plsc.subcore_barrier()
    pltpu.sync_copy(acc.at[pl.ds(sid * 640, 640)],
                    out_hbm.at[cid, pl.ds(sid * 640, 640)])


_deg_call = pl.kernel(
    _deg_body,
    out_type=jax.ShapeDtypeStruct((NC, NPAD), jnp.float32),
    mesh=_mesh,
    scratch_types=[
        pltpu.VMEM((2, 8, D), jnp.int32),
        pltpu.VMEM((2, 8, D), jnp.float32),
        pltpu.VMEM((640,), jnp.float32),
        pltpu.VMEM_SHARED((NPAD,), jnp.float32),
        pltpu.SemaphoreType.DMA,
        pltpu.SemaphoreType.DMA,
    ],
)


NQ = ER // 8
NW = NC * NS
QMAX = (NQ + NW - 1) // NW
NTAIL = ER - NQ * 8
_DO_MUL = True
_DO_SCAT = True


HR = 64
NQ2 = NE // (8 * HR)
QMAX2 = (NQ2 + NW - 1) // NW


def _prop_body(g_hbm, srcq_hbm, dstq_hbm, ewq_hbm,
               out_hbm, srcv, dstv, ewv, rows, acc, gsem, ssem, isem):
    cid = lax.axis_index("c")
    sid = lax.axis_index("s")
    w = sid * NC + cid
    nq = jnp.where(w < NQ2 % NW, QMAX2, NQ2 // NW)

    def iload(kb):
        s = kb % 2
        q = kb * NW + w
        pltpu.async_copy(srcq_hbm.at[q], srcv.at[s], isem)
        pltpu.async_copy(dstq_hbm.at[q], dstv.at[s], isem)
        pltpu.async_copy(ewq_hbm.at[q], ewv.at[s], isem)

    def iwait(kb):
        s = kb % 2
        q = kb * NW + w
        pltpu.make_async_copy(srcq_hbm.at[q], srcv.at[s], isem).wait()
        pltpu.make_async_copy(dstq_hbm.at[q], dstv.at[s], isem).wait()
        pltpu.make_async_copy(ewq_hbm.at[q], ewv.at[s], isem).wait()

    iload(0)

    def zero(i, _):
        for t in range(8):
            rows[0, i, pl.ds(t * 16, 16)] = jnp.zeros((16,), jnp.float32)
        return ()
    lax.fori_loop(0, HR, zero, ())
    for c in range(9):
        pltpu.sync_copy(rows.at[0, pl.ds(0, 64)],
                        acc.at[pl.ds(sid * NZT + c * 64, 64)])
    pltpu.sync_copy(rows.at[0, pl.ds(0, 48)],
                    acc.at[pl.ds(sid * NZT + 576, 48)])

    @pl.when(sid == 0)
    def _():
        pltpu.sync_copy(rows.at[0, pl.ds(0, 16)], acc.at[pl.ds(NS * NZT, 16)])

    iwait(0)
    plsc.subcore_barrier()

    def gather_start(s, j, b):
        pltpu.async_copy(g_hbm.at[srcv.at[s, j]], rows.at[b], gsem)

    def gather_wait(b):
        pltpu.make_async_copy(g_hbm.at[srcv.at[0, 0]], rows.at[b],
                              gsem).wait()

    def scatter_start(s, j, b):
        pltpu.async_copy(rows.at[b], acc.at[dstv.at[s, j]], ssem, add=True)

    def scatter_wait():
        pltpu.make_async_copy(rows.at[0], acc.at[dstv.at[0, 0]], ssem).wait()

    def multiply(s, j, b):
        @plsc.parallel_loop(0, HR // 16, unroll=4)
        def _(g):
            wv = ewv[s, j, pl.ds(g * 16, 16)]
            for jj in range(16):
                w_ = wv[jj]
                k = g * 16 + jj
                for t in range(8):
                    rows[b, k, pl.ds(t * 16, 16)] = (
                        rows[b, k, pl.ds(t * 16, 16)] * w_)

    gather_start(0, 0, 0)
    gather_start(0, 1, 1)

    def batch(kb, _):
        @pl.when(kb < nq)
        def _():
            bc = kb % 2

            def body(j, _):
                i = kb * 8 + j
                b = lax.rem(i, 3)
                gather_wait(b)
                multiply(bc, j, b)
                scatter_start(bc, j, b)

                @pl.when(jnp.logical_and(j == 1, kb + 1 < nq))
                def _():
                    iload(kb + 1)

                @pl.when(i >= 1)
                def _():
                    scatter_wait()

                @pl.when(j < 6)
                def _():
                    gather_start(bc, j + 2, lax.rem(i + 2, 3))

                @pl.when(jnp.logical_and(j == 6, kb + 1 < nq))
                def _():
                    iwait(kb + 1)
                    gather_start(1 - bc, 0, lax.rem(i + 2, 3))

                @pl.when(jnp.logical_and(j == 7, kb + 1 < nq))
                def _():
                    gather_start(1 - bc, 1, lax.rem(i + 2, 3))
                return ()
            lax.fori_loop(0, 8, body, ())
        return ()
    lax.fori_loop(0, QMAX2, batch, ())
    scatter_wait()

    plsc.subcore_barrier()
    pltpu.sync_copy(acc.at[pl.ds(sid * NZT, NZT)],
                    out_hbm.at[cid, pl.ds(sid * NZT, NZT)])

    @pl.when(sid == 0)
    def _():
        pltpu.sync_copy(acc.at[pl.ds(NS * NZT, 16)],
                        out_hbm.at[cid, pl.ds(NS * NZT, 16)])


_prop_call = pl.kernel(
    _prop_body,
    out_type=jax.ShapeDtypeStruct((NC, NN, D), jnp.float32),
    mesh=_mesh,
    scratch_types=[
        pltpu.VMEM((2, 8, HR), jnp.int32),
        pltpu.VMEM((2, 8, HR), jnp.int32),
        pltpu.VMEM((2, 8, HR), jnp.float32),
        pltpu.VMEM((3, HR, D), jnp.float32),
        pltpu.VMEM_SHARED((NN, D), jnp.float32),
        pltpu.SemaphoreType.DMA,
        pltpu.SemaphoreType.DMA,
        pltpu.SemaphoreType.DMA,
    ],
)


_NB = 1000
_GRID = NN // _NB


def _tc1_body(degp_ref, x_ref, w1_ref, dinv_ref, g1_ref):
    deg = 1.0 + degp_ref[:, 0] + degp_ref[:, 1]
    r = lax.rsqrt(jnp.maximum(deg, 1e-12))
    r = jnp.where(deg > 0, r, 0.0)[:, None]
    dinv_ref[...] = r
    h = jnp.dot(x_ref[...], w1_ref[...], preferred_element_type=jnp.float32)
    g1_ref[...] = h * r


def _tc2_body(acc_ref, g1_ref, dinv_ref, b1_ref, w2_ref, g2_ref):
    r = dinv_ref[...]
    h = (acc_ref[0] + acc_ref[1] + g1_ref[...]) * r + b1_ref[...][None, :]
    z = jnp.maximum(h, 0.0)
    g2_ref[...] = jnp.dot(z, w2_ref[...],
                          preferred_element_type=jnp.float32) * r


def _tc3_body(acc_ref, g2_ref, dinv_ref, b2_ref, wl_ref, bl_ref, out_ref):
    h = (acc_ref[0] + acc_ref[1] + g2_ref[...]) * dinv_ref[...] \
        + b2_ref[...][None, :]
    out_ref[...] = jnp.dot(h, wl_ref[...],
                           preferred_element_type=jnp.float32) \
        + bl_ref[...][None, :]


_node_spec = pl.BlockSpec((_NB, D), lambda i: (i, 0))
_dinv_spec = pl.BlockSpec((_NB, 1), lambda i: (i, 0))
_w_spec = pl.BlockSpec((D, D), lambda i: (0, 0))
_b_spec = pl.BlockSpec((D,), lambda i: (0,))
_acc_spec = pl.BlockSpec((NC, _NB, D), lambda i: (0, i, 0))

_tc1_call = pl.pallas_call(
    _tc1_body,
    grid=(_GRID,),
    in_specs=[pl.BlockSpec((_NB, NC), lambda i: (i, 0)), _node_spec, _w_spec],
    out_specs=[_dinv_spec, _node_spec],
    out_shape=[jax.ShapeDtypeStruct((NN, 1), jnp.float32),
               jax.ShapeDtypeStruct((NN, D), jnp.float32)],
)

_tc2_call = pl.pallas_call(
    _tc2_body,
    grid=(_GRID,),
    in_specs=[_acc_spec, _node_spec, _dinv_spec, _b_spec, _w_spec],
    out_specs=_node_spec,
    out_shape=jax.ShapeDtypeStruct((NN, D), jnp.float32),
)

_tc3_call = pl.pallas_call(
    _tc3_body,
    grid=(_GRID,),
    in_specs=[_acc_spec, _node_spec, _dinv_spec, _b_spec, _w_spec, _b_spec],
    out_specs=_node_spec,
    out_shape=jax.ShapeDtypeStruct((NN, D), jnp.float32),
)


def kernel(x, edge_index, edge_weight, W1, b1, W2, b2, Wl, bl):
    ei = edge_index.astype(jnp.int32)
    src = ei[0].reshape(ER, D)
    dst = ei[1].reshape(ER, D)
    ew = edge_weight.astype(jnp.float32).reshape(ER, D)
    dstq = dst[:NQ * 8].reshape(NQ, 8, D)
    ewq = ew[:NQ * 8].reshape(NQ, 8, D)
    dt, et = dst[NQ * 8:], ew[NQ * 8:]
    srcq2 = src.reshape(NQ2, 8, HR)
    dstq2 = dst.reshape(NQ2, 8, HR)
    ewq2 = ew.reshape(NQ2, 8, HR)

    degp = _deg_call(dstq, ewq, dt, et)[:, :NN].T
    dinv, g1 = _tc1_call(degp, x, W1)
    acc1 = _prop_call(g1, srcq2, dstq2, ewq2)
    g2 = _tc2_call(acc1, g1, dinv, b1, W2)
    acc2 = _prop_call(g2, srcq2, dstq2, ewq2)
    return _tc3_call(acc2, g2, dinv, b2, Wl, bl)


# speedup vs baseline: 1.9969x; 1.0003x over previous
import jax
import jax.numpy as jnp
from jax import lax
from jax.experimental import pallas as pl
from jax.experimental.pallas import tpu as pltpu
from jax.experimental.pallas import tpu_sc as plsc

NN = 10000
NE = 320000
D = 128
NC = 2
NS = 16
ER = NE // D
ERH = ER // NC
NPAD = 10240
NZT = 624

_mesh = plsc.VectorSubcoreMesh(
    core_axis_name="c", subcore_axis_name="s", num_cores=NC, num_subcores=NS)


def _deg_body(dstq_hbm, ewq_hbm, dt_hbm, et_hbm, out_hbm,
              dstv, ewv, zv, acc, isem, ssem):
    cid = lax.axis_index("c")
    sid = lax.axis_index("s")
    w = sid * NC + cid
    nq = jnp.where(w < NQ % NW, QMAX, NQ // NW)

    def iload(kb):
        s = kb % 2
        q = kb * NW + w
        pltpu.async_copy(dstq_hbm.at[q], dstv.at[s], isem)
        pltpu.async_copy(ewq_hbm.at[q], ewv.at[s], isem)

    def iwait(kb):
        s = kb % 2
        q = kb * NW + w
        pltpu.make_async_copy(dstq_hbm.at[q], dstv.at[s], isem).wait()
        pltpu.make_async_copy(ewq_hbm.at[q], ewv.at[s], isem).wait()

    iload(0)

    def zero(i, _):
        zv[pl.ds(i * 16, 16)] = jnp.zeros((16,), jnp.float32)
        return ()
    lax.fori_loop(0, 40, zero, ())
    pltpu.sync_copy(zv, acc.at[pl.ds(sid * 640, 640)])
    iwait(0)
    plsc.subcore_barrier()

    def body(kb, _):
        @pl.when(kb < nq)
        def _():
            s = kb % 2

            @pl.when(kb + 1 < nq)
            def _():
                iload(kb + 1)
            for j in range(8):
                pltpu.async_copy(ewv.at[s, j], acc.at[dstv.at[s, j]], ssem,
                                 add=True)
            for j in range(8):
                pltpu.make_async_copy(ewv.at[s, j], acc.at[dstv.at[s, j]],
                                      ssem).wait()

            @pl.when(kb + 1 < nq)
            def _():
                iwait(kb + 1)
        return ()
    lax.fori_loop(0, QMAX, body, ())

    @pl.when(w < NTAIL)
    def _():
        pltpu.sync_copy(dt_hbm.at[w], dstv.at[0, 0])
        pltpu.sync_copy(et_hbm.at[w], ewv.at[0, 0])
        pltpu.sync_copy(ewv.at[0, 0], acc.at[dstv.at[0, 0]], add=True)

    plsc.subcore_barrier()
    pltpu.sync_copy(acc.at[pl.ds(sid * 640, 640)],
                    out_hbm.at[cid, pl.ds(sid * 640, 640)])


_deg_call = pl.kernel(
    _deg_body,
    out_type=jax.ShapeDtypeStruct((NC, NPAD), jnp.float32),
    mesh=_mesh,
    scratch_types=[
        pltpu.VMEM((2, 8, D), jnp.int32),
        pltpu.VMEM((2, 8, D), jnp.float32),
        pltpu.VMEM((640,), jnp.float32),
        pltpu.VMEM_SHARED((NPAD,), jnp.float32),
        pltpu.SemaphoreType.DMA,
        pltpu.SemaphoreType.DMA,
    ],
)


NQ = ER // 8
NW = NC * NS
QMAX = (NQ + NW - 1) // NW
NTAIL = ER - NQ * 8


HR = 64
NQ2 = NE // (8 * HR)
QMAX2 = (NQ2 + NW - 1) // NW


def _prop_body(g_hbm, srcq_hbm, dstq_hbm, ewq_hbm,
               out_hbm, srcv, dstv, ewv, rows, acc, gsem, ssem, isem):
    cid = lax.axis_index("c")
    sid = lax.axis_index("s")
    w = sid * NC + cid
    nq = jnp.where(w < NQ2 % NW, QMAX2, NQ2 // NW)

    def iload(kb):
        s = kb % 2
        q = kb * NW + w
        pltpu.async_copy(srcq_hbm.at[q], srcv.at[s], isem)
        pltpu.async_copy(dstq_hbm.at[q], dstv.at[s], isem)
        pltpu.async_copy(ewq_hbm.at[q], ewv.at[s], isem)

    def iwait(kb):
        s = kb % 2
        q = kb * NW + w
        pltpu.make_async_copy(srcq_hbm.at[q], srcv.at[s], isem).wait()
        pltpu.make_async_copy(dstq_hbm.at[q], dstv.at[s], isem).wait()
        pltpu.make_async_copy(ewq_hbm.at[q], ewv.at[s], isem).wait()

    iload(0)

    def zero(i, _):
        for t in range(8):
            rows[0, i, pl.ds(t * 16, 16)] = jnp.zeros((16,), jnp.float32)
        return ()
    lax.fori_loop(0, HR, zero, ())
    for c in range(9):
        pltpu.sync_copy(rows.at[0, pl.ds(0, 64)],
                        acc.at[pl.ds(sid * NZT + c * 64, 64)])
    pltpu.sync_copy(rows.at[0, pl.ds(0, 48)],
                    acc.at[pl.ds(sid * NZT + 576, 48)])

    @pl.when(sid == 0)
    def _():
        pltpu.sync_copy(rows.at[0, pl.ds(0, 16)], acc.at[pl.ds(NS * NZT, 16)])

    iwait(0)
    plsc.subcore_barrier()

    def gather_start(s, j, b):
        pltpu.async_copy(g_hbm.at[srcv.at[s, j]], rows.at[b], gsem)

    def gather_wait(b):
        pltpu.make_async_copy(g_hbm.at[srcv.at[0, 0]], rows.at[b],
                              gsem).wait()

    def scatter_start(s, j, b):
        pltpu.async_copy(rows.at[b], acc.at[dstv.at[s, j]], ssem, add=True)

    def scatter_wait():
        pltpu.make_async_copy(rows.at[0], acc.at[dstv.at[0, 0]], ssem).wait()

    def multiply(s, j, b):
        @plsc.parallel_loop(0, HR // 16, unroll=4)
        def _(g):
            wv = ewv[s, j, pl.ds(g * 16, 16)]
            for jj in range(16):
                w_ = wv[jj]
                k = g * 16 + jj
                for t in range(8):
                    rows[b, k, pl.ds(t * 16, 16)] = (
                        rows[b, k, pl.ds(t * 16, 16)] * w_)

    gather_start(0, 0, 0)
    gather_start(0, 1, 1)

    def batch(kb, _):
        @pl.when(kb < nq)
        def _():
            bc = kb % 2

            def body(j, _):
                i = kb * 8 + j
                b = lax.rem(i, 3)
                gather_wait(b)
                multiply(bc, j, b)
                scatter_start(bc, j, b)

                @pl.when(jnp.logical_and(j == 1, kb + 1 < nq))
                def _():
                    iload(kb + 1)

                @pl.when(i >= 1)
                def _():
                    scatter_wait()

                @pl.when(j < 6)
                def _():
                    gather_start(bc, j + 2, lax.rem(i + 2, 3))

                @pl.when(jnp.logical_and(j == 6, kb + 1 < nq))
                def _():
                    iwait(kb + 1)
                    gather_start(1 - bc, 0, lax.rem(i + 2, 3))

                @pl.when(jnp.logical_and(j == 7, kb + 1 < nq))
                def _():
                    gather_start(1 - bc, 1, lax.rem(i + 2, 3))
                return ()
            lax.fori_loop(0, 8, body, ())
        return ()
    lax.fori_loop(0, QMAX2, batch, ())
    scatter_wait()

    plsc.subcore_barrier()
    pltpu.sync_copy(acc.at[pl.ds(sid * NZT, NZT)],
                    out_hbm.at[cid, pl.ds(sid * NZT, NZT)])

    @pl.when(sid == 0)
    def _():
        pltpu.sync_copy(acc.at[pl.ds(NS * NZT, 16)],
                        out_hbm.at[cid, pl.ds(NS * NZT, 16)])


_prop_call = pl.kernel(
    _prop_body,
    out_type=jax.ShapeDtypeStruct((NC, NN, D), jnp.float32),
    mesh=_mesh,
    scratch_types=[
        pltpu.VMEM((2, 8, HR), jnp.int32),
        pltpu.VMEM((2, 8, HR), jnp.int32),
        pltpu.VMEM((2, 8, HR), jnp.float32),
        pltpu.VMEM((3, HR, D), jnp.float32),
        pltpu.VMEM_SHARED((NN, D), jnp.float32),
        pltpu.SemaphoreType.DMA,
        pltpu.SemaphoreType.DMA,
        pltpu.SemaphoreType.DMA,
    ],
)


_NB = 1000
_GRID = NN // _NB


def _tc1_body(degp_ref, x_ref, w1_ref, dinv_ref, g1_ref):
    deg = 1.0 + degp_ref[:, 0] + degp_ref[:, 1]
    r = lax.rsqrt(jnp.maximum(deg, 1e-12))
    r = jnp.where(deg > 0, r, 0.0)[:, None]
    dinv_ref[...] = r
    h = jnp.dot(x_ref[...], w1_ref[...], preferred_element_type=jnp.float32)
    g1_ref[...] = h * r


def _tc2_body(acc_ref, g1_ref, dinv_ref, b1_ref, w2_ref, g2_ref):
    r = dinv_ref[...]
    h = (acc_ref[0] + acc_ref[1] + g1_ref[...]) * r + b1_ref[...][None, :]
    z = jnp.maximum(h, 0.0)
    g2_ref[...] = jnp.dot(z, w2_ref[...],
                          preferred_element_type=jnp.float32) * r


def _tc3_body(acc_ref, g2_ref, dinv_ref, b2_ref, wl_ref, bl_ref, out_ref):
    h = (acc_ref[0] + acc_ref[1] + g2_ref[...]) * dinv_ref[...] \
        + b2_ref[...][None, :]
    out_ref[...] = jnp.dot(h, wl_ref[...],
                           preferred_element_type=jnp.float32) \
        + bl_ref[...][None, :]


_node_spec = pl.BlockSpec((_NB, D), lambda i: (i, 0))
_dinv_spec = pl.BlockSpec((_NB, 1), lambda i: (i, 0))
_w_spec = pl.BlockSpec((D, D), lambda i: (0, 0))
_b_spec = pl.BlockSpec((D,), lambda i: (0,))
_acc_spec = pl.BlockSpec((NC, _NB, D), lambda i: (0, i, 0))

_tc1_call = pl.pallas_call(
    _tc1_body,
    grid=(_GRID,),
    in_specs=[pl.BlockSpec((_NB, NC), lambda i: (i, 0)), _node_spec, _w_spec],
    out_specs=[_dinv_spec, _node_spec],
    out_shape=[jax.ShapeDtypeStruct((NN, 1), jnp.float32),
               jax.ShapeDtypeStruct((NN, D), jnp.float32)],
)

_tc2_call = pl.pallas_call(
    _tc2_body,
    grid=(_GRID,),
    in_specs=[_acc_spec, _node_spec, _dinv_spec, _b_spec, _w_spec],
    out_specs=_node_spec,
    out_shape=jax.ShapeDtypeStruct((NN, D), jnp.float32),
)

_tc3_call = pl.pallas_call(
    _tc3_body,
    grid=(_GRID,),
    in_specs=[_acc_spec, _node_spec, _dinv_spec, _b_spec, _w_spec, _b_spec],
    out_specs=_node_spec,
    out_shape=jax.ShapeDtypeStruct((NN, D), jnp.float32),
)


def kernel(x, edge_index, edge_weight, W1, b1, W2, b2, Wl, bl):
    ei = edge_index.astype(jnp.int32)
    src = ei[0].reshape(ER, D)
    dst = ei[1].reshape(ER, D)
    ew = edge_weight.astype(jnp.float32).reshape(ER, D)
    dstq = dst[:NQ * 8].reshape(NQ, 8, D)
    ewq = ew[:NQ * 8].reshape(NQ, 8, D)
    dt, et = dst[NQ * 8:], ew[NQ * 8:]
    srcq2 = src.reshape(NQ2, 8, HR)
    dstq2 = dst.reshape(NQ2, 8, HR)
    ewq2 = ew.reshape(NQ2, 8, HR)

    degp = _deg_call(dstq, ewq, dt, et)[:, :NN].T
    dinv, g1 = _tc1_call(degp, x, W1)
    acc1 = _prop_call(g1, srcq2, dstq2, ewq2)
    g2 = _tc2_call(acc1, g1, dinv, b1, W2)
    acc2 = _prop_call(g2, srcq2, dstq2, ewq2)
    return _tc3_call(acc2, g2, dinv, b2, Wl, bl)
